# baseline (speedup 1.0000x reference)
import functools

import jax
import jax.numpy as jnp
from jax import lax
from jax.experimental import pallas as pl
from jax.experimental.pallas import tpu as pltpu
from jax.experimental.pallas import tpu_sc as plsc

N = 10000
NPAD = 10240
E = 320000
EP = 344064
ET = EP // 16
CH = 128
NCHUNK = ET // CH
WEP = EP // 32
NS_NODES = NPAD // 16


def _make_tc_matmul(cin_half, cout):
    half = cout // 2
    BM = 512

    def body(hlo_ref, hhi_ref, wlo_ref, whi_ref, att_ref,
             xlo_ref, xhi_ref, aux_ref, cm_ref):
        y = jnp.dot(hlo_ref[...], wlo_ref[...],
                    preferred_element_type=jnp.float32)
        y = y + jnp.dot(hhi_ref[...], whi_ref[...],
                        preferred_element_type=jnp.float32)
        xlo_ref[...] = y[:, :half]
        xhi_ref[...] = y[:, half:]
        ab = jnp.dot(y, att_ref[...], preferred_element_type=jnp.float32)
        aux_ref[...] = jnp.concatenate(
            [ab, jnp.zeros((BM, 126), jnp.float32)], axis=1)
        ms = jnp.max(ab[:, 0])
        md = jnp.max(ab[:, 1])
        i = pl.program_id(0)

        @pl.when(i == 0)
        def _():
            cm_ref[0, 0] = ms
            cm_ref[0, 1] = md

        @pl.when(i > 0)
        def _():
            cm_ref[0, 0] = jnp.maximum(cm_ref[0, 0], ms)
            cm_ref[0, 1] = jnp.maximum(cm_ref[0, 1], md)

    return pl.pallas_call(
        body,
        grid=(NPAD // BM,),
        in_specs=[
            pl.BlockSpec((BM, cin_half), lambda i: (i, 0)),
            pl.BlockSpec((BM, cin_half), lambda i: (i, 0)),
            pl.BlockSpec((cin_half, cout), lambda i: (0, 0)),
            pl.BlockSpec((cin_half, cout), lambda i: (0, 0)),
            pl.BlockSpec((cout, 2), lambda i: (0, 0)),
        ],
        out_specs=[
            pl.BlockSpec((BM, half), lambda i: (i, 0)),
            pl.BlockSpec((BM, half), lambda i: (i, 0)),
            pl.BlockSpec((BM, 128), lambda i: (i, 0)),
            pl.BlockSpec((1, 2), lambda i: (0, 0), memory_space=pltpu.SMEM),
        ],
        out_shape=[
            jax.ShapeDtypeStruct((NPAD, half), jnp.float32),
            jax.ShapeDtypeStruct((NPAD, half), jnp.float32),
            jax.ShapeDtypeStruct((NPAD, 128), jnp.float32),
            jax.ShapeDtypeStruct((1, 2), jnp.float32),
        ],
    )


def _make_sc_alpha():
    mesh = plsc.VectorSubcoreMesh(core_axis_name="c", subcore_axis_name="s")

    @functools.partial(
        pl.kernel,
        out_type=[
            jax.ShapeDtypeStruct((EP,), jnp.float32),
            jax.ShapeDtypeStruct((2, NPAD), jnp.float32),
        ],
        mesh=mesh,
        compiler_params=pltpu.CompilerParams(needs_layout_passes=False),
        scratch_types=[
            pltpu.VMEM((NPAD,), jnp.float32),
            pltpu.VMEM((NPAD,), jnp.float32),
            pltpu.VMEM((WEP,), jnp.int32),
            pltpu.VMEM((WEP,), jnp.int32),
            pltpu.VMEM((WEP,), jnp.float32),
            pltpu.VMEM((NPAD,), jnp.float32),
            pltpu.VMEM((16,), jnp.float32),
            pltpu.VMEM((NS_NODES,), jnp.float32),
            pltpu.VMEM((NS_NODES,), jnp.float32),
            pltpu.VMEM_SHARED((16, NPAD), jnp.float32),
        ],
    )
    def k(src_h, dst_h, asrc_h, adst_h, cb_h, ex_hbm, denom_hbm,
          asrc_v, adst_v, src_v, dst_v, ex_v, denom_v, cb_v, dn_v, tmp_v,
          denom_sh):
        c = lax.axis_index("c")
        s = lax.axis_index("s")
        zero16 = jnp.zeros((16,), jnp.float32)
        base = (c * 16 + s) * WEP

        pltpu.sync_copy(src_h.at[pl.ds(base, WEP)], src_v)
        pltpu.sync_copy(dst_h.at[pl.ds(base, WEP)], dst_v)
        pltpu.sync_copy(asrc_h, asrc_v)
        pltpu.sync_copy(adst_h, adst_v)
        pltpu.sync_copy(cb_h, cb_v)
        cb = cb_v[...]

        def zd(i, carry):
            denom_v[pl.ds(i * 16, 16)] = zero16
            return carry

        lax.fori_loop(0, NPAD // 16, zd, 0)

        def p1(i, carry):
            s16 = src_v[pl.ds(i * 16, 16)]
            d16 = dst_v[pl.ds(i * 16, 16)]
            a = plsc.load_gather(asrc_v, [s16]) + plsc.load_gather(adst_v, [d16])
            a = jnp.where(a > 0, a, 0.2 * a) - cb
            e = jnp.exp(a)
            ex_v[pl.ds(i * 16, 16)] = e
            plsc.addupdate_scatter(denom_v, [d16], e)
            return carry

        lax.fori_loop(0, WEP // 16, p1, 0)
        pltpu.sync_copy(ex_v, ex_hbm.at[pl.ds(base, WEP)])
        pltpu.sync_copy(denom_v, denom_sh.at[s])
        plsc.subcore_barrier()

        nbase = s * NS_NODES

        def zdn(i, carry):
            dn_v[pl.ds(i * 16, 16)] = zero16
            return carry

        lax.fori_loop(0, NS_NODES // 16, zdn, 0)
        for w in range(16):
            pltpu.sync_copy(denom_sh.at[w, pl.ds(nbase, NS_NODES)], tmp_v)

            def radd(i, carry):
                dn_v[pl.ds(i * 16, 16)] = (dn_v[pl.ds(i * 16, 16)]
                                           + tmp_v[pl.ds(i * 16, 16)])
                return carry

            lax.fori_loop(0, NS_NODES // 16, radd, 0)
        pltpu.sync_copy(dn_v, denom_hbm.at[c, pl.ds(nbase, NS_NODES)])

    return k


BLKC = 8
BLKE = BLKC * CH
NBLK = NCHUNK // BLKC


def _make_sc_accum(half, do_relu):
    mesh = plsc.VectorSubcoreMesh(core_axis_name="c", subcore_axis_name="s")

    @functools.partial(
        pl.kernel,
        out_type=[
            jax.ShapeDtypeStruct((NPAD, half), jnp.float32),
            jax.ShapeDtypeStruct((NPAD, half), jnp.float32),
        ],
        mesh=mesh,
        compiler_params=pltpu.CompilerParams(needs_layout_passes=False),
        scratch_types=[
            pltpu.VMEM((BLKE,), jnp.int32),
            pltpu.VMEM((BLKC, CH), jnp.int32),
            pltpu.VMEM((BLKE,), jnp.float32),
            pltpu.VMEM((CH, half), jnp.float32),
            pltpu.VMEM((CH, half), jnp.float32),
            pltpu.VMEM((half,), jnp.float32),
            pltpu.VMEM((NS_NODES,), jnp.float32),
            pltpu.VMEM((NS_NODES,), jnp.float32),
            pltpu.VMEM_SHARED((NPAD, half), jnp.float32),
            pltpu.SemaphoreType.DMA,
            pltpu.SemaphoreType.DMA,
        ],
    )
    def k(xl_lo, xl_hi, src_h, dst2_h, ex_h, denom_h, blo_h, bhi_h,
          out_lo, out_hi,
          srcb_v, dstb_v, exb_v, rows_a, rows_b, bias_v, dn_v, tmp_v,
          acc_sh, sem_a, sem_b):
        c = lax.axis_index("c")
        s = lax.axis_index("s")
        zero16 = jnp.zeros((16,), jnp.float32)
        nbase = s * NS_NODES
        rows = [rows_a, rows_b]
        sems = [sem_a, sem_b]

        @pl.when(c == 0)
        def _():
            pltpu.sync_copy(blo_h, bias_v)

        @pl.when(c == 1)
        def _():
            pltpu.sync_copy(bhi_h, bias_v)

        def zr(r, carry):
            for j in range(half // 16):
                rows_a[r, pl.ds(j * 16, 16)] = zero16
            return carry

        lax.fori_loop(0, CH, zr, 0)
        for q in range(NS_NODES // CH):
            pltpu.sync_copy(rows_a, acc_sh.at[pl.ds(nbase + q * CH, CH)])

        plsc.subcore_barrier()

        def phase2(xl_h):
            def blk_body(b, carry):
                ebase = s * ET + b * BLKE
                pltpu.sync_copy(src_h.at[pl.ds(ebase, BLKE)], srcb_v)
                pltpu.sync_copy(dst2_h.at[pl.ds(s * NCHUNK + b * BLKC, BLKC)],
                                dstb_v)
                pltpu.sync_copy(ex_h.at[pl.ds(ebase, BLKE)], exb_v)

                handles = [None, None]
                handles[0] = pltpu.async_copy(
                    xl_h.at[srcb_v.at[pl.ds(0, CH)]], rows_a, sem_a)
                for ci in range(BLKC):
                    p = ci % 2
                    if ci + 1 < BLKC:
                        handles[1 - p] = pltpu.async_copy(
                            xl_h.at[srcb_v.at[pl.ds((ci + 1) * CH, CH)]],
                            rows[1 - p], sems[1 - p])
                    handles[p].wait()
                    rv = rows[p]

                    def scale(r, carry2):
                        exr = plsc.load_gather(
                            exb_v, [jnp.full((16,), ci * CH, jnp.int32) + r])
                        for j in range(half // 16):
                            rv[r, pl.ds(j * 16, 16)] = (
                                rv[r, pl.ds(j * 16, 16)] * exr)
                        return carry2

                    lax.fori_loop(0, CH, scale, 0)
                    pltpu.sync_copy(rv, acc_sh.at[dstb_v.at[ci]], add=True)
                return carry

            lax.fori_loop(0, NBLK, blk_body, 0)

        @pl.when(c == 0)
        def _():
            phase2(xl_lo)

        @pl.when(c == 1)
        def _():
            phase2(xl_hi)

        plsc.subcore_barrier()

        pltpu.sync_copy(denom_h.at[0, pl.ds(nbase, NS_NODES)], dn_v)
        pltpu.sync_copy(denom_h.at[1, pl.ds(nbase, NS_NODES)], tmp_v)

        def inv(i, carry):
            dsum = dn_v[pl.ds(i * 16, 16)] + tmp_v[pl.ds(i * 16, 16)]
            dn_v[pl.ds(i * 16, 16)] = 1.0 / (dsum + 1e-16)
            return carry

        lax.fori_loop(0, NS_NODES // 16, inv, 0)
        bias_vecs = [bias_v[pl.ds(j * 16, 16)] for j in range(half // 16)]

        def epilogue(out_h):
            for q in range(NS_NODES // CH):
                pltpu.sync_copy(acc_sh.at[pl.ds(nbase + q * CH, CH)], rows_a)

                def nr(r, carry):
                    dn = plsc.load_gather(
                        dn_v, [jnp.full((16,), q * CH, jnp.int32) + r])
                    for j in range(half // 16):
                        val = rows_a[r, pl.ds(j * 16, 16)] * dn + bias_vecs[j]
                        if do_relu:
                            val = jnp.maximum(val, 0.0)
                        rows_a[r, pl.ds(j * 16, 16)] = val
                    return carry

                lax.fori_loop(0, CH, nr, 0)
                pltpu.sync_copy(rows_a, out_h.at[pl.ds(nbase + q * CH, CH)])

        @pl.when(c == 0)
        def _():
            epilogue(out_lo)

        @pl.when(c == 1)
        def _():
            epilogue(out_hi)

    return k


def _gat_layer(h_lo, h_hi, src, dst, W, att_src, att_dst, b, do_relu):
    cin = h_lo.shape[1] * 2
    cout = W.shape[1]
    half = cout // 2
    tc = _make_tc_matmul(cin // 2, cout)
    att2 = jnp.stack([att_src, att_dst], axis=1)
    xl_lo, xl_hi, aux, cm = tc(h_lo, h_hi, W[: cin // 2], W[cin // 2:], att2)
    s_bound = cm[0, 0] + cm[0, 1]
    c_scalar = jnp.maximum(s_bound, 0.2 * s_bound)
    cb = jnp.full((16,), c_scalar, jnp.float32)
    asrc = aux[:, 0] + 0.0
    adst = aux[:, 1] + 0.0
    ex, denom = _make_sc_alpha()(src, dst, asrc, adst, cb)
    out_lo, out_hi = _make_sc_accum(half, do_relu)(
        xl_lo, xl_hi, src, dst.reshape(EP // CH, CH), ex, denom,
        b[:half], b[half:])
    return out_lo, out_hi


def kernel(x, edge_index, W0, att_src0, att_dst0, b0,
           W1, att_src1, att_dst1, b1, W2, att_src2, att_dst2, b2):
    xp = jnp.zeros((NPAD, x.shape[1]), jnp.float32).at[:N].set(x)
    h_lo, h_hi = xp[:, : x.shape[1] // 2], xp[:, x.shape[1] // 2:]

    loop = jnp.arange(N, dtype=jnp.int32)
    pad = jnp.full((EP - E - N,), NPAD - 1, jnp.int32)
    src = jnp.concatenate([edge_index[0], loop, pad])
    dst = jnp.concatenate([edge_index[1], loop, pad])

    h_lo, h_hi = _gat_layer(h_lo, h_hi, src, dst, W0, att_src0, att_dst0,
                            b0, True)
    h_lo, h_hi = _gat_layer(h_lo, h_hi, src, dst, W1, att_src1, att_dst1,
                            b1, True)
    oc = W2.shape[1]
    W2p = jnp.concatenate([W2, jnp.zeros((W2.shape[0], oc), jnp.float32)], 1)
    z = jnp.zeros((oc,), jnp.float32)
    h_lo, _ = _gat_layer(h_lo, h_hi, src, dst, W2p,
                         jnp.concatenate([att_src2, z]),
                         jnp.concatenate([att_dst2, z]),
                         jnp.concatenate([b2, z]), False)
    return h_lo[:N]


# speedup vs baseline: 1.4075x; 1.4075x over previous
import functools

import jax
import jax.numpy as jnp
from jax import lax
from jax.experimental import pallas as pl
from jax.experimental.pallas import tpu as pltpu
from jax.experimental.pallas import tpu_sc as plsc

N = 10000
NPAD = 10240
E = 320000
EP = 331776
ET = EP // 16
CH = 128
NCHUNK = ET // CH
WEP = EP // 32
NS_NODES = NPAD // 16

_GD = lax.GatherDimensionNumbers(
    offset_dims=(), collapsed_slice_dims=(0,), start_index_map=(0,))


def _lane(v16, r):
    idx = jnp.full((16, 1), r, jnp.int32)
    return lax.gather(v16, idx, _GD, (1,),
                      mode=lax.GatherScatterMode.PROMISE_IN_BOUNDS)


def _make_tc_matmul(cin_half, cout):
    half = cout // 2
    BM = 512

    def body(hlo_ref, hhi_ref, wlo_ref, whi_ref, att_ref,
             xlo_ref, xhi_ref, aux_ref, cm_ref):
        y = jnp.dot(hlo_ref[...], wlo_ref[...],
                    preferred_element_type=jnp.float32)
        y = y + jnp.dot(hhi_ref[...], whi_ref[...],
                        preferred_element_type=jnp.float32)
        xlo_ref[...] = y[:, :half]
        xhi_ref[...] = y[:, half:]
        ab = jnp.dot(y, att_ref[...], preferred_element_type=jnp.float32)
        aux_ref[...] = jnp.concatenate(
            [ab, jnp.zeros((BM, 126), jnp.float32)], axis=1)
        ms = jnp.max(ab[:, 0])
        md = jnp.max(ab[:, 1])
        i = pl.program_id(0)

        @pl.when(i == 0)
        def _():
            cm_ref[0, 0] = ms
            cm_ref[0, 1] = md

        @pl.when(i > 0)
        def _():
            cm_ref[0, 0] = jnp.maximum(cm_ref[0, 0], ms)
            cm_ref[0, 1] = jnp.maximum(cm_ref[0, 1], md)

    return pl.pallas_call(
        body,
        grid=(NPAD // BM,),
        in_specs=[
            pl.BlockSpec((BM, cin_half), lambda i: (i, 0)),
            pl.BlockSpec((BM, cin_half), lambda i: (i, 0)),
            pl.BlockSpec((cin_half, cout), lambda i: (0, 0)),
            pl.BlockSpec((cin_half, cout), lambda i: (0, 0)),
            pl.BlockSpec((cout, 2), lambda i: (0, 0)),
        ],
        out_specs=[
            pl.BlockSpec((BM, half), lambda i: (i, 0)),
            pl.BlockSpec((BM, half), lambda i: (i, 0)),
            pl.BlockSpec((BM, 128), lambda i: (i, 0)),
            pl.BlockSpec((1, 2), lambda i: (0, 0), memory_space=pltpu.SMEM),
        ],
        out_shape=[
            jax.ShapeDtypeStruct((NPAD, half), jnp.float32),
            jax.ShapeDtypeStruct((NPAD, half), jnp.float32),
            jax.ShapeDtypeStruct((NPAD, 128), jnp.float32),
            jax.ShapeDtypeStruct((1, 2), jnp.float32),
        ],
    )


def _make_sc_alpha():
    mesh = plsc.VectorSubcoreMesh(core_axis_name="c", subcore_axis_name="s")

    @functools.partial(
        pl.kernel,
        out_type=[
            jax.ShapeDtypeStruct((EP,), jnp.float32),
            jax.ShapeDtypeStruct((2, NPAD), jnp.float32),
        ],
        mesh=mesh,
        compiler_params=pltpu.CompilerParams(needs_layout_passes=False),
        scratch_types=[
            pltpu.VMEM((NPAD,), jnp.float32),
            pltpu.VMEM((NPAD,), jnp.float32),
            pltpu.VMEM((WEP,), jnp.int32),
            pltpu.VMEM((WEP,), jnp.int32),
            pltpu.VMEM((WEP,), jnp.float32),
            pltpu.VMEM((NPAD,), jnp.float32),
            pltpu.VMEM((16,), jnp.float32),
            pltpu.VMEM((NS_NODES,), jnp.float32),
            pltpu.VMEM((NS_NODES,), jnp.float32),
            pltpu.VMEM_SHARED((16, NPAD), jnp.float32),
        ],
    )
    def k(src_h, dst_h, asrc_h, adst_h, cb_h, ex_hbm, denom_hbm,
          asrc_v, adst_v, src_v, dst_v, ex_v, denom_v, cb_v, dn_v, tmp_v,
          denom_sh):
        c = lax.axis_index("c")
        s = lax.axis_index("s")
        zero16 = jnp.zeros((16,), jnp.float32)
        base = (c * 16 + s) * WEP

        pltpu.sync_copy(src_h.at[pl.ds(base, WEP)], src_v)
        pltpu.sync_copy(dst_h.at[pl.ds(base, WEP)], dst_v)
        pltpu.sync_copy(asrc_h, asrc_v)
        pltpu.sync_copy(adst_h, adst_v)
        pltpu.sync_copy(cb_h, cb_v)
        cb = cb_v[...]

        def zd(i, carry):
            denom_v[pl.ds(i * 16, 16)] = zero16
            return carry

        lax.fori_loop(0, NPAD // 16, zd, 0)

        def p1(i, carry):
            s16 = src_v[pl.ds(i * 16, 16)]
            d16 = dst_v[pl.ds(i * 16, 16)]
            a = plsc.load_gather(asrc_v, [s16]) + plsc.load_gather(adst_v, [d16])
            a = jnp.where(a > 0, a, 0.2 * a) - cb
            e = jnp.exp(a)
            ex_v[pl.ds(i * 16, 16)] = e
            plsc.addupdate_scatter(denom_v, [d16], e)
            return carry

        lax.fori_loop(0, WEP // 16, p1, 0)
        pltpu.sync_copy(ex_v, ex_hbm.at[pl.ds(base, WEP)])
        pltpu.sync_copy(denom_v, denom_sh.at[s])
        plsc.subcore_barrier()

        nbase = s * NS_NODES

        def zdn(i, carry):
            dn_v[pl.ds(i * 16, 16)] = zero16
            return carry

        lax.fori_loop(0, NS_NODES // 16, zdn, 0)
        for w in range(16):
            pltpu.sync_copy(denom_sh.at[w, pl.ds(nbase, NS_NODES)], tmp_v)

            def radd(i, carry):
                dn_v[pl.ds(i * 16, 16)] = (dn_v[pl.ds(i * 16, 16)]
                                           + tmp_v[pl.ds(i * 16, 16)])
                return carry

            lax.fori_loop(0, NS_NODES // 16, radd, 0)
        pltpu.sync_copy(dn_v, denom_hbm.at[c, pl.ds(nbase, NS_NODES)])

    return k


def _make_sc_accum(half, do_relu):
    mesh = plsc.VectorSubcoreMesh(core_axis_name="c", subcore_axis_name="s")

    @functools.partial(
        pl.kernel,
        out_type=[
            jax.ShapeDtypeStruct((NPAD, half), jnp.float32),
            jax.ShapeDtypeStruct((NPAD, half), jnp.float32),
        ],
        mesh=mesh,
        compiler_params=pltpu.CompilerParams(needs_layout_passes=False),
        scratch_types=[
            pltpu.VMEM((CH,), jnp.int32),
            pltpu.VMEM((CH,), jnp.int32),
            pltpu.VMEM((CH,), jnp.float32),
            pltpu.VMEM((CH, half), jnp.float32),
            pltpu.VMEM((half,), jnp.float32),
            pltpu.VMEM((NS_NODES,), jnp.float32),
            pltpu.VMEM((NS_NODES,), jnp.float32),
            pltpu.VMEM_SHARED((NPAD, half), jnp.float32),
        ],
    )
    def k(xl_lo, xl_hi, src_h, dst_h, ex_h, denom_h, blo_h, bhi_h,
          out_lo, out_hi,
          srcc_v, dstc_v, exc_v, rows_a, bias_v, dn_v, tmp_v, acc_sh):
        c = lax.axis_index("c")
        s = lax.axis_index("s")
        zero16 = jnp.zeros((16,), jnp.float32)
        nbase = s * NS_NODES

        @pl.when(c == 0)
        def _():
            pltpu.sync_copy(blo_h, bias_v)

        @pl.when(c == 1)
        def _():
            pltpu.sync_copy(bhi_h, bias_v)

        def zr(r, carry):
            for j in range(half // 16):
                rows_a[r, pl.ds(j * 16, 16)] = zero16
            return carry

        lax.fori_loop(0, CH, zr, 0)
        for q in range(NS_NODES // CH):
            pltpu.sync_copy(rows_a, acc_sh.at[pl.ds(nbase + q * CH, CH)])

        plsc.subcore_barrier()

        def phase2(xl_h):
            def chunk(i, carry):
                ebase = s * ET + i * CH
                pltpu.sync_copy(src_h.at[pl.ds(ebase, CH)], srcc_v)
                pltpu.sync_copy(dst_h.at[pl.ds(ebase, CH)], dstc_v)
                pltpu.sync_copy(ex_h.at[pl.ds(ebase, CH)], exc_v)
                pltpu.sync_copy(xl_h.at[srcc_v], rows_a)

                def scale16(g, carry2):
                    ex16 = exc_v[pl.ds(g * 16, 16)]
                    for r in range(16):
                        er = _lane(ex16, r)
                        row = g * 16 + r
                        for j in range(half // 16):
                            rows_a[row, pl.ds(j * 16, 16)] = (
                                rows_a[row, pl.ds(j * 16, 16)] * er)
                    return carry2

                lax.fori_loop(0, CH // 16, scale16, 0)
                pltpu.sync_copy(rows_a, acc_sh.at[dstc_v], add=True)
                return carry

            lax.fori_loop(0, NCHUNK, chunk, 0)

        @pl.when(c == 0)
        def _():
            phase2(xl_lo)

        @pl.when(c == 1)
        def _():
            phase2(xl_hi)

        plsc.subcore_barrier()

        pltpu.sync_copy(denom_h.at[0, pl.ds(nbase, NS_NODES)], dn_v)
        pltpu.sync_copy(denom_h.at[1, pl.ds(nbase, NS_NODES)], tmp_v)

        def inv(i, carry):
            dsum = dn_v[pl.ds(i * 16, 16)] + tmp_v[pl.ds(i * 16, 16)]
            dn_v[pl.ds(i * 16, 16)] = 1.0 / (dsum + 1e-16)
            return carry

        lax.fori_loop(0, NS_NODES // 16, inv, 0)
        bias_vecs = [bias_v[pl.ds(j * 16, 16)] for j in range(half // 16)]

        def epilogue(out_h):
            for q in range(NS_NODES // CH):
                pltpu.sync_copy(acc_sh.at[pl.ds(nbase + q * CH, CH)], rows_a)

                def nr16(g, carry):
                    dn16 = dn_v[pl.ds(q * CH + g * 16, 16)]
                    for r in range(16):
                        dnr = _lane(dn16, r)
                        row = g * 16 + r
                        for j in range(half // 16):
                            val = (rows_a[row, pl.ds(j * 16, 16)] * dnr
                                   + bias_vecs[j])
                            if do_relu:
                                val = jnp.maximum(val, 0.0)
                            rows_a[row, pl.ds(j * 16, 16)] = val
                    return carry

                lax.fori_loop(0, CH // 16, nr16, 0)
                pltpu.sync_copy(rows_a, out_h.at[pl.ds(nbase + q * CH, CH)])

        @pl.when(c == 0)
        def _():
            epilogue(out_lo)

        @pl.when(c == 1)
        def _():
            epilogue(out_hi)

    return k


def _gat_layer(h_lo, h_hi, src, dst, W, att_src, att_dst, b, do_relu):
    cin = h_lo.shape[1] * 2
    cout = W.shape[1]
    half = cout // 2
    tc = _make_tc_matmul(cin // 2, cout)
    att2 = jnp.stack([att_src, att_dst], axis=1)
    xl_lo, xl_hi, aux, cm = tc(h_lo, h_hi, W[: cin // 2], W[cin // 2:], att2)
    s_bound = cm[0, 0] + cm[0, 1]
    c_scalar = jnp.maximum(s_bound, 0.2 * s_bound)
    cb = jnp.full((16,), c_scalar, jnp.float32)
    asrc = aux[:, 0] + 0.0
    adst = aux[:, 1] + 0.0
    ex, denom = _make_sc_alpha()(src, dst, asrc, adst, cb)
    out_lo, out_hi = _make_sc_accum(half, do_relu)(
        xl_lo, xl_hi, src, dst, ex, denom, b[:half], b[half:])
    return out_lo, out_hi


def kernel(x, edge_index, W0, att_src0, att_dst0, b0,
           W1, att_src1, att_dst1, b1, W2, att_src2, att_dst2, b2):
    xp = jnp.zeros((NPAD, x.shape[1]), jnp.float32).at[:N].set(x)
    h_lo, h_hi = xp[:, : x.shape[1] // 2], xp[:, x.shape[1] // 2:]

    loop = jnp.arange(N, dtype=jnp.int32)
    pad = jnp.full((EP - E - N,), NPAD - 1, jnp.int32)
    src = jnp.concatenate([edge_index[0], loop, pad])
    dst = jnp.concatenate([edge_index[1], loop, pad])

    h_lo, h_hi = _gat_layer(h_lo, h_hi, src, dst, W0, att_src0, att_dst0,
                            b0, True)
    h_lo, h_hi = _gat_layer(h_lo, h_hi, src, dst, W1, att_src1, att_dst1,
                            b1, True)
    oc = W2.shape[1]
    W2p = jnp.concatenate([W2, jnp.zeros((W2.shape[0], oc), jnp.float32)], 1)
    z = jnp.zeros((oc,), jnp.float32)
    h_lo, _ = _gat_layer(h_lo, h_hi, src, dst, W2p,
                         jnp.concatenate([att_src2, z]),
                         jnp.concatenate([att_dst2, z]),
                         jnp.concatenate([b2, z]), False)
    return h_lo[:N]


# speedup vs baseline: 1.9035x; 1.3524x over previous
import functools

import jax
import jax.numpy as jnp
from jax import lax
from jax.experimental import pallas as pl
from jax.experimental.pallas import tpu as pltpu
from jax.experimental.pallas import tpu_sc as plsc

N = 10000
NPAD = 10240
E = 320000
EP = 331776
ET = EP // 16
CH = 128
NCHUNK = ET // CH
WEP = EP // 32
NS_NODES = NPAD // 16

_GD = lax.GatherDimensionNumbers(
    offset_dims=(), collapsed_slice_dims=(0,), start_index_map=(0,))


def _lane(v16, r):
    idx = jnp.full((16, 1), r, jnp.int32)
    return lax.gather(v16, idx, _GD, (1,),
                      mode=lax.GatherScatterMode.PROMISE_IN_BOUNDS)


def _make_tc_matmul(cin_half, cout):
    half = cout // 2
    BM = 512

    def body(hlo_ref, hhi_ref, wlo_ref, whi_ref, att_ref,
             xlo_ref, xhi_ref, aux_ref, cm_ref):
        y = jnp.dot(hlo_ref[...], wlo_ref[...],
                    preferred_element_type=jnp.float32)
        y = y + jnp.dot(hhi_ref[...], whi_ref[...],
                        preferred_element_type=jnp.float32)
        xlo_ref[...] = y[:, :half]
        xhi_ref[...] = y[:, half:]
        ab = jnp.dot(y, att_ref[...], preferred_element_type=jnp.float32)
        aux_ref[...] = jnp.concatenate(
            [ab, jnp.zeros((BM, 126), jnp.float32)], axis=1)
        ms = jnp.max(ab[:, 0])
        md = jnp.max(ab[:, 1])
        i = pl.program_id(0)

        @pl.when(i == 0)
        def _():
            cm_ref[0, 0] = ms
            cm_ref[0, 1] = md

        @pl.when(i > 0)
        def _():
            cm_ref[0, 0] = jnp.maximum(cm_ref[0, 0], ms)
            cm_ref[0, 1] = jnp.maximum(cm_ref[0, 1], md)

    return pl.pallas_call(
        body,
        grid=(NPAD // BM,),
        in_specs=[
            pl.BlockSpec((BM, cin_half), lambda i: (i, 0)),
            pl.BlockSpec((BM, cin_half), lambda i: (i, 0)),
            pl.BlockSpec((cin_half, cout), lambda i: (0, 0)),
            pl.BlockSpec((cin_half, cout), lambda i: (0, 0)),
            pl.BlockSpec((cout, 2), lambda i: (0, 0)),
        ],
        out_specs=[
            pl.BlockSpec((BM, half), lambda i: (i, 0)),
            pl.BlockSpec((BM, half), lambda i: (i, 0)),
            pl.BlockSpec((BM, 128), lambda i: (i, 0)),
            pl.BlockSpec((1, 2), lambda i: (0, 0), memory_space=pltpu.SMEM),
        ],
        out_shape=[
            jax.ShapeDtypeStruct((NPAD, half), jnp.float32),
            jax.ShapeDtypeStruct((NPAD, half), jnp.float32),
            jax.ShapeDtypeStruct((NPAD, 128), jnp.float32),
            jax.ShapeDtypeStruct((1, 2), jnp.float32),
        ],
    )


def _make_sc_alpha():
    mesh = plsc.VectorSubcoreMesh(core_axis_name="c", subcore_axis_name="s")

    @functools.partial(
        pl.kernel,
        out_type=[
            jax.ShapeDtypeStruct((EP,), jnp.float32),
            jax.ShapeDtypeStruct((2, NPAD), jnp.float32),
        ],
        mesh=mesh,
        compiler_params=pltpu.CompilerParams(needs_layout_passes=False),
        scratch_types=[
            pltpu.VMEM((NPAD,), jnp.float32),
            pltpu.VMEM((NPAD,), jnp.float32),
            pltpu.VMEM((WEP,), jnp.int32),
            pltpu.VMEM((WEP,), jnp.int32),
            pltpu.VMEM((WEP,), jnp.float32),
            pltpu.VMEM((NPAD,), jnp.float32),
            pltpu.VMEM((16,), jnp.float32),
            pltpu.VMEM((NS_NODES,), jnp.float32),
            pltpu.VMEM((NS_NODES,), jnp.float32),
            pltpu.VMEM_SHARED((16, NPAD), jnp.float32),
        ],
    )
    def k(src_h, dst_h, asrc_h, adst_h, cb_h, ex_hbm, denom_hbm,
          asrc_v, adst_v, src_v, dst_v, ex_v, denom_v, cb_v, dn_v, tmp_v,
          denom_sh):
        c = lax.axis_index("c")
        s = lax.axis_index("s")
        zero16 = jnp.zeros((16,), jnp.float32)
        base = (c * 16 + s) * WEP

        pltpu.sync_copy(src_h.at[pl.ds(base, WEP)], src_v)
        pltpu.sync_copy(dst_h.at[pl.ds(base, WEP)], dst_v)
        pltpu.sync_copy(asrc_h, asrc_v)
        pltpu.sync_copy(adst_h, adst_v)
        pltpu.sync_copy(cb_h, cb_v)
        cb = cb_v[...]

        def zd(i, carry):
            denom_v[pl.ds(i * 16, 16)] = zero16
            return carry

        lax.fori_loop(0, NPAD // 16, zd, 0)

        def p1(i, carry):
            s16 = src_v[pl.ds(i * 16, 16)]
            d16 = dst_v[pl.ds(i * 16, 16)]
            a = plsc.load_gather(asrc_v, [s16]) + plsc.load_gather(adst_v, [d16])
            a = jnp.where(a > 0, a, 0.2 * a) - cb
            e = jnp.exp(a)
            ex_v[pl.ds(i * 16, 16)] = e
            plsc.addupdate_scatter(denom_v, [d16], e)
            return carry

        lax.fori_loop(0, WEP // 16, p1, 0)
        pltpu.sync_copy(ex_v, ex_hbm.at[pl.ds(base, WEP)])
        pltpu.sync_copy(denom_v, denom_sh.at[s])
        plsc.subcore_barrier()

        nbase = s * NS_NODES

        def zdn(i, carry):
            dn_v[pl.ds(i * 16, 16)] = zero16
            return carry

        lax.fori_loop(0, NS_NODES // 16, zdn, 0)
        for w in range(16):
            pltpu.sync_copy(denom_sh.at[w, pl.ds(nbase, NS_NODES)], tmp_v)

            def radd(i, carry):
                dn_v[pl.ds(i * 16, 16)] = (dn_v[pl.ds(i * 16, 16)]
                                           + tmp_v[pl.ds(i * 16, 16)])
                return carry

            lax.fori_loop(0, NS_NODES // 16, radd, 0)
        pltpu.sync_copy(dn_v, denom_hbm.at[c, pl.ds(nbase, NS_NODES)])

    return k


def _make_sc_accum(half, do_relu):
    mesh = plsc.VectorSubcoreMesh(core_axis_name="c", subcore_axis_name="s")

    @functools.partial(
        pl.kernel,
        out_type=[
            jax.ShapeDtypeStruct((NPAD, half), jnp.float32),
            jax.ShapeDtypeStruct((NPAD, half), jnp.float32),
        ],
        mesh=mesh,
        compiler_params=pltpu.CompilerParams(needs_layout_passes=False),
        scratch_types=[
            pltpu.VMEM((CH,), jnp.int32),
            pltpu.VMEM((CH,), jnp.int32),
            pltpu.VMEM((CH,), jnp.int32),
            pltpu.VMEM((CH,), jnp.int32),
            pltpu.VMEM((CH,), jnp.float32),
            pltpu.VMEM((CH,), jnp.float32),
            pltpu.VMEM((CH, half), jnp.float32),
            pltpu.VMEM((CH, half), jnp.float32),
            pltpu.VMEM((half,), jnp.float32),
            pltpu.VMEM((NS_NODES,), jnp.float32),
            pltpu.VMEM((NS_NODES,), jnp.float32),
            pltpu.VMEM_SHARED((NPAD, half), jnp.float32),
            pltpu.SemaphoreType.DMA,
            pltpu.SemaphoreType.DMA,
        ],
    )
    def k(xl_lo, xl_hi, src_h, dst_h, ex_h, denom_h, blo_h, bhi_h,
          out_lo, out_hi,
          srcc_a, srcc_b, dstc_a, dstc_b, exc_a, exc_b, rows_a, rows_b,
          bias_v, dn_v, tmp_v, acc_sh, sem_a, sem_b):
        c = lax.axis_index("c")
        s = lax.axis_index("s")
        zero16 = jnp.zeros((16,), jnp.float32)
        nbase = s * NS_NODES

        @pl.when(c == 0)
        def _():
            pltpu.sync_copy(blo_h, bias_v)

        @pl.when(c == 1)
        def _():
            pltpu.sync_copy(bhi_h, bias_v)

        def zr(r, carry):
            for j in range(half // 16):
                rows_a[r, pl.ds(j * 16, 16)] = zero16
            return carry

        lax.fori_loop(0, CH, zr, 0)
        for q in range(NS_NODES // CH):
            pltpu.sync_copy(rows_a, acc_sh.at[pl.ds(nbase + q * CH, CH)])

        plsc.subcore_barrier()

        srccs = [srcc_a, srcc_b]
        dstcs = [dstc_a, dstc_b]
        excs = [exc_a, exc_b]
        rowss = [rows_a, rows_b]
        sems = [sem_a, sem_b]

        def phase2(xl_h):
            def stage_and_start(i, b):
                ebase = s * ET + i * CH
                pltpu.sync_copy(src_h.at[pl.ds(ebase, CH)], srccs[b])
                pltpu.sync_copy(dst_h.at[pl.ds(ebase, CH)], dstcs[b])
                pltpu.sync_copy(ex_h.at[pl.ds(ebase, CH)], excs[b])
                pltpu.async_copy(xl_h.at[srccs[b]], rowss[b], sems[b])

            def consume(b):
                pltpu.make_async_copy(
                    xl_h.at[srccs[b]], rowss[b], sems[b]).wait()
                rv = rowss[b]
                exc_v = excs[b]

                def scale16(g, carry2):
                    ex16 = exc_v[pl.ds(g * 16, 16)]
                    for r in range(16):
                        er = _lane(ex16, r)
                        row = g * 16 + r
                        for j in range(half // 16):
                            rv[row, pl.ds(j * 16, 16)] = (
                                rv[row, pl.ds(j * 16, 16)] * er)
                    return carry2

                lax.fori_loop(0, CH // 16, scale16, 0)
                pltpu.sync_copy(rv, acc_sh.at[dstcs[b]], add=True)

            stage_and_start(0, 0)

            def pair(p, carry):
                stage_and_start(2 * p + 1, 1)
                consume(0)

                @pl.when(p < NCHUNK // 2 - 1)
                def _():
                    stage_and_start(2 * p + 2, 0)

                consume(1)
                return carry

            lax.fori_loop(0, NCHUNK // 2, pair, 0)

        @pl.when(c == 0)
        def _():
            phase2(xl_lo)

        @pl.when(c == 1)
        def _():
            phase2(xl_hi)

        plsc.subcore_barrier()

        pltpu.sync_copy(denom_h.at[0, pl.ds(nbase, NS_NODES)], dn_v)
        pltpu.sync_copy(denom_h.at[1, pl.ds(nbase, NS_NODES)], tmp_v)

        def inv(i, carry):
            dsum = dn_v[pl.ds(i * 16, 16)] + tmp_v[pl.ds(i * 16, 16)]
            dn_v[pl.ds(i * 16, 16)] = 1.0 / (dsum + 1e-16)
            return carry

        lax.fori_loop(0, NS_NODES // 16, inv, 0)
        bias_vecs = [bias_v[pl.ds(j * 16, 16)] for j in range(half // 16)]

        def epilogue(out_h):
            for q in range(NS_NODES // CH):
                pltpu.sync_copy(acc_sh.at[pl.ds(nbase + q * CH, CH)], rows_a)

                def nr16(g, carry):
                    dn16 = dn_v[pl.ds(q * CH + g * 16, 16)]
                    for r in range(16):
                        dnr = _lane(dn16, r)
                        row = g * 16 + r
                        for j in range(half // 16):
                            val = (rows_a[row, pl.ds(j * 16, 16)] * dnr
                                   + bias_vecs[j])
                            if do_relu:
                                val = jnp.maximum(val, 0.0)
                            rows_a[row, pl.ds(j * 16, 16)] = val
                    return carry

                lax.fori_loop(0, CH // 16, nr16, 0)
                pltpu.sync_copy(rows_a, out_h.at[pl.ds(nbase + q * CH, CH)])

        @pl.when(c == 0)
        def _():
            epilogue(out_lo)

        @pl.when(c == 1)
        def _():
            epilogue(out_hi)

    return k


def _gat_layer(h_lo, h_hi, src, dst, W, att_src, att_dst, b, do_relu):
    cin = h_lo.shape[1] * 2
    cout = W.shape[1]
    half = cout // 2
    tc = _make_tc_matmul(cin // 2, cout)
    att2 = jnp.stack([att_src, att_dst], axis=1)
    xl_lo, xl_hi, aux, cm = tc(h_lo, h_hi, W[: cin // 2], W[cin // 2:], att2)
    s_bound = cm[0, 0] + cm[0, 1]
    c_scalar = jnp.maximum(s_bound, 0.2 * s_bound)
    cb = jnp.full((16,), c_scalar, jnp.float32)
    asrc = aux[:, 0] + 0.0
    adst = aux[:, 1] + 0.0
    ex, denom = _make_sc_alpha()(src, dst, asrc, adst, cb)
    out_lo, out_hi = _make_sc_accum(half, do_relu)(
        xl_lo, xl_hi, src, dst, ex, denom, b[:half], b[half:])
    return out_lo, out_hi


def kernel(x, edge_index, W0, att_src0, att_dst0, b0,
           W1, att_src1, att_dst1, b1, W2, att_src2, att_dst2, b2):
    xp = jnp.zeros((NPAD, x.shape[1]), jnp.float32).at[:N].set(x)
    h_lo, h_hi = xp[:, : x.shape[1] // 2], xp[:, x.shape[1] // 2:]

    loop = jnp.arange(N, dtype=jnp.int32)
    pad = jnp.full((EP - E - N,), NPAD - 1, jnp.int32)
    src = jnp.concatenate([edge_index[0], loop, pad])
    dst = jnp.concatenate([edge_index[1], loop, pad])

    h_lo, h_hi = _gat_layer(h_lo, h_hi, src, dst, W0, att_src0, att_dst0,
                            b0, True)
    h_lo, h_hi = _gat_layer(h_lo, h_hi, src, dst, W1, att_src1, att_dst1,
                            b1, True)
    oc = W2.shape[1]
    W2p = jnp.concatenate([W2, jnp.zeros((W2.shape[0], oc), jnp.float32)], 1)
    z = jnp.zeros((oc,), jnp.float32)
    h_lo, _ = _gat_layer(h_lo, h_hi, src, dst, W2p,
                         jnp.concatenate([att_src2, z]),
                         jnp.concatenate([att_dst2, z]),
                         jnp.concatenate([b2, z]), False)
    return h_lo[:N]
